# Initial kernel scaffold; baseline (speedup 1.0000x reference)
import jax
import jax.numpy as jnp
from jax.experimental import pallas as pl


def kernel(x, edge_index, Ws, bs):
    raise NotImplementedError("write your pallas kernel here")


# speedup vs baseline: 2.0477x; 2.0477x over previous
import functools

import jax
import jax.numpy as jnp
from jax import lax
from jax.experimental import pallas as pl
from jax.experimental.pallas import tpu as pltpu
from jax.experimental.pallas import tpu_sc as plsc

N = 10000
D = 128
E = 320000
K = 10
NUM_LAYERS = 8

NC = 2
NS = 16
NW = NC * NS
CH = 128
NCH = 80
EPT = CH * NCH
EPAD = EPT * NW
NP = 10240
RPT = NP // NS
TRASH = N

_mesh = plsc.VectorSubcoreMesh(core_axis_name="c", subcore_axis_name="s")


@functools.partial(
    pl.kernel,
    out_type=jax.ShapeDtypeStruct((NC, NP, D), jnp.float32),
    mesh=_mesh,
    scratch_types=[
        pltpu.VMEM((NCH, CH), jnp.int32),
        pltpu.VMEM((NCH, CH), jnp.int32),
        pltpu.VMEM((CH, D), jnp.float32),
        pltpu.VMEM_SHARED((NP, D), jnp.float32),
        pltpu.SemaphoreType.DMA,
    ],
)
def _sc_prop(xs, gidx, sidx, out, gi_v, si_v, rows, acc, gsem):
    cid = lax.axis_index("c")
    sid = lax.axis_index("s")
    t = cid * NS + sid

    pltpu.sync_copy(gidx.at[t], gi_v)
    pltpu.sync_copy(sidx.at[t], si_v)

    def _zr(i, c):
        for j in range(D // 16):
            rows[i, pl.ds(j * 16, 16)] = jnp.zeros((16,), jnp.float32)
        return c

    lax.fori_loop(0, CH, _zr, 0)
    for i in range(RPT // CH):
        pltpu.sync_copy(rows, acc.at[pl.ds(sid * RPT + i * CH, CH)])
    plsc.subcore_barrier()

    def _body(j, c):
        pltpu.async_copy(xs.at[gi_v.at[j]], rows, gsem).wait()
        pltpu.sync_copy(rows, acc.at[si_v.at[j]], add=True)
        return c

    lax.fori_loop(0, NCH, _body, 0)
    plsc.subcore_barrier()
    pltpu.sync_copy(acc.at[pl.ds(sid * RPT, RPT)],
                    out.at[cid, pl.ds(sid * RPT, RPT)])


BR = 1000
GR = N // BR

_row_spec = pl.BlockSpec((BR, D), lambda i: (i, 0))
_w_spec = pl.BlockSpec((D, D), lambda i: (0, 0))
_z_spec = pl.BlockSpec((NC, BR, D), lambda i: (0, i, 0))


def _prep_body(z_ref, dinv_ref):
    deg = z_ref[0] + z_ref[1]
    dinv_ref[...] = jnp.where(deg > 0.0, 1.0 / jnp.sqrt(jnp.maximum(deg, 1.0)), 0.0)


def _tc_prep(z):
    return pl.pallas_call(
        _prep_body,
        grid=(GR,),
        in_specs=[_z_spec],
        out_specs=_row_spec,
        out_shape=jax.ShapeDtypeStruct((N, D), jnp.float32),
    )(z)


def _k0_body(h_ref, w_ref, dinv_ref, out_ref, sh_ref):
    h = h_ref[...]
    out_ref[...] = jnp.dot(h, w_ref[...], preferred_element_type=jnp.float32)
    sh_ref[...] = h * dinv_ref[...]


def _tc_k0(h, w0, dinv):
    return pl.pallas_call(
        _k0_body,
        grid=(GR,),
        in_specs=[_row_spec, _w_spec, _row_spec],
        out_specs=[_row_spec, _row_spec],
        out_shape=[jax.ShapeDtypeStruct((N, D), jnp.float32)] * 2,
    )(h, w0, dinv)


def _step_body(alpha, beta, z_ref, txm2_ref, out_ref, w_ref, dinv_ref,
               outn_ref, tx_ref, sx_ref):
    dinv = dinv_ref[...]
    tx = (-alpha) * dinv * (z_ref[0] + z_ref[1])
    if beta:
        tx = tx - beta * txm2_ref[...]
    outn_ref[...] = out_ref[...] + jnp.dot(tx, w_ref[...],
                                           preferred_element_type=jnp.float32)
    tx_ref[...] = tx
    sx_ref[...] = dinv * tx


def _tc_step(z, txm2, out, w, dinv, alpha, beta):
    return pl.pallas_call(
        functools.partial(_step_body, alpha, beta),
        grid=(GR,),
        in_specs=[_z_spec, _row_spec, _row_spec, _w_spec, _row_spec],
        out_specs=[_row_spec, _row_spec, _row_spec],
        out_shape=[jax.ShapeDtypeStruct((N, D), jnp.float32)] * 3,
    )(z, txm2, out, w, dinv)


def _fin_body(relu, out_ref, b_ref, h_ref):
    v = out_ref[...] + b_ref[...]
    h_ref[...] = jnp.maximum(v, 0.0) if relu else v


def _tc_fin(out, b, relu):
    return pl.pallas_call(
        functools.partial(_fin_body, relu),
        grid=(GR,),
        in_specs=[_row_spec, pl.BlockSpec((1, D), lambda i: (0, 0))],
        out_specs=_row_spec,
        out_shape=jax.ShapeDtypeStruct((N, D), jnp.float32),
    )(out, b)


def kernel(x, edge_index, Ws, bs):
    src = edge_index[0]
    dst = edge_index[1]
    mask = src != dst
    pad = EPAD - E
    trash = jnp.full((pad,), TRASH, jnp.int32)
    gidx = jnp.concatenate([src, jnp.zeros((pad,), jnp.int32)]).reshape(NW, NCH, CH)
    sidx_deg = jnp.concatenate(
        [jnp.where(mask, src, TRASH), trash]).reshape(NW, NCH, CH)
    sidx = jnp.concatenate(
        [jnp.where(mask, dst, TRASH), trash]).reshape(NW, NCH, CH)
    gidx_zero = jnp.zeros((NW, NCH, CH), jnp.int32)

    ones_tab = jnp.ones((8, D), jnp.float32)
    deg_rows = _sc_prop(ones_tab, gidx_zero, sidx_deg)
    dinv = _tc_prep(deg_rows)

    h = x
    for l in range(NUM_LAYERS):
        W = Ws[l]
        out, sh = _tc_k0(h, W[0], dinv)
        z = _sc_prop(sh, gidx, sidx)
        out, tx_prev, sx_prev = _tc_step(z, h, out, W[1], dinv, 1.0, 0.0)
        tx_pp = h
        for k in range(2, K):
            z = _sc_prop(sx_prev, gidx, sidx)
            out, tx_k, sx_k = _tc_step(z, tx_pp, out, W[k], dinv, 2.0, 1.0)
            tx_pp, tx_prev, sx_prev = tx_prev, tx_k, sx_k
        h = _tc_fin(out, bs[l][None, :], relu=(l < NUM_LAYERS - 1))
    return h


# speedup vs baseline: 2.3110x; 1.1286x over previous
import functools

import jax
import jax.numpy as jnp
from jax import lax
from jax.experimental import pallas as pl
from jax.experimental.pallas import tpu as pltpu
from jax.experimental.pallas import tpu_sc as plsc

N = 10000
D = 128
E = 320000
K = 10
NUM_LAYERS = 8

NC = 2
NS = 16
NW = NC * NS
CH = 128
NCH = 80
EPT = CH * NCH
EPAD = EPT * NW
NP = 10112
RPT = NP // NS
TRASH = N

_mesh = plsc.VectorSubcoreMesh(core_axis_name="c", subcore_axis_name="s")

NBUF = 2
HALVES = 2
HNCH = NCH // HALVES


@functools.partial(
    pl.kernel,
    out_type=jax.ShapeDtypeStruct((NC, NP, D), jnp.float32),
    mesh=_mesh,
    scratch_types=[
        pltpu.VMEM((HNCH, CH), jnp.int32),
        pltpu.VMEM((HNCH, CH), jnp.int32),
        pltpu.VMEM((NBUF, CH, D), jnp.float32),
        pltpu.VMEM_SHARED((NP, D), jnp.float32),
        pltpu.SemaphoreType.DMA,
    ],
)
def _sc_prop(xs, gidx, sidx, out, gi_v, si_v, rows, acc, gsem):
    cid = lax.axis_index("c")
    sid = lax.axis_index("s")
    t = cid * NS + sid

    def _zr(i, c):
        for j in range(D // 16):
            rows[0, i, pl.ds(j * 16, 16)] = jnp.zeros((16,), jnp.float32)
        return c

    lax.fori_loop(0, CH, _zr, 0)
    for i in range(RPT // CH):
        pltpu.sync_copy(rows.at[0], acc.at[pl.ds(sid * RPT + i * CH, CH)])
    pltpu.sync_copy(rows.at[0, pl.ds(0, RPT - (RPT // CH) * CH)],
                    acc.at[pl.ds(sid * RPT + (RPT // CH) * CH,
                                 RPT - (RPT // CH) * CH)])
    plsc.subcore_barrier()

    def _gather(j, b):
        return pltpu.async_copy(xs.at[gi_v.at[j]], rows.at[b], gsem)

    def _gather_wait(j, b):
        pltpu.make_async_copy(xs.at[gi_v.at[j]], rows.at[b], gsem).wait()

    def _scatter(j, b):
        pltpu.sync_copy(rows.at[b], acc.at[si_v.at[j]], add=True)

    for h in range(HALVES):
        pltpu.sync_copy(gidx.at[t, pl.ds(h * HNCH, HNCH)], gi_v)
        pltpu.sync_copy(sidx.at[t, pl.ds(h * HNCH, HNCH)], si_v)
        _gather(0, 0)

        def _body(i, c):
            j = i * NBUF
            _gather_wait(j, 0)
            _gather(j + 1, 1)
            _scatter(j, 0)
            nj = j + 2

            @pl.when(nj < HNCH)
            def _():
                _gather(nj, 0)

            _gather_wait(j + 1, 1)
            _scatter(j + 1, 1)
            return c

        lax.fori_loop(0, HNCH // NBUF, _body, 0)

    plsc.subcore_barrier()
    pltpu.sync_copy(acc.at[pl.ds(sid * RPT, RPT)],
                    out.at[cid, pl.ds(sid * RPT, RPT)])


BR = 1000
GR = N // BR

_row_spec = pl.BlockSpec((BR, D), lambda i: (i, 0))
_w_spec = pl.BlockSpec((D, D), lambda i: (0, 0))
_z_spec = pl.BlockSpec((NC, BR, D), lambda i: (0, i, 0))


def _prep_body(z_ref, dinv_ref):
    deg = z_ref[0] + z_ref[1]
    dinv_ref[...] = jnp.where(deg > 0.0, 1.0 / jnp.sqrt(jnp.maximum(deg, 1.0)), 0.0)


def _tc_prep(z):
    return pl.pallas_call(
        _prep_body,
        grid=(GR,),
        in_specs=[_z_spec],
        out_specs=_row_spec,
        out_shape=jax.ShapeDtypeStruct((N, D), jnp.float32),
    )(z)


def _k0_body(h_ref, w_ref, dinv_ref, out_ref, sh_ref):
    h = h_ref[...]
    out_ref[...] = jnp.dot(h, w_ref[...], preferred_element_type=jnp.float32)
    sh_ref[...] = h * dinv_ref[...]


def _tc_k0(h, w0, dinv):
    return pl.pallas_call(
        _k0_body,
        grid=(GR,),
        in_specs=[_row_spec, _w_spec, _row_spec],
        out_specs=[_row_spec, _row_spec],
        out_shape=[jax.ShapeDtypeStruct((N, D), jnp.float32)] * 2,
    )(h, w0, dinv)


def _step_body(alpha, beta, z_ref, txm2_ref, out_ref, w_ref, dinv_ref,
               outn_ref, tx_ref, sx_ref):
    dinv = dinv_ref[...]
    tx = (-alpha) * dinv * (z_ref[0] + z_ref[1])
    if beta:
        tx = tx - beta * txm2_ref[...]
    outn_ref[...] = out_ref[...] + jnp.dot(tx, w_ref[...],
                                           preferred_element_type=jnp.float32)
    tx_ref[...] = tx
    sx_ref[...] = dinv * tx


def _tc_step(z, txm2, out, w, dinv, alpha, beta):
    return pl.pallas_call(
        functools.partial(_step_body, alpha, beta),
        grid=(GR,),
        in_specs=[_z_spec, _row_spec, _row_spec, _w_spec, _row_spec],
        out_specs=[_row_spec, _row_spec, _row_spec],
        out_shape=[jax.ShapeDtypeStruct((N, D), jnp.float32)] * 3,
    )(z, txm2, out, w, dinv)


def _fin_body(relu, out_ref, b_ref, h_ref):
    v = out_ref[...] + b_ref[...]
    h_ref[...] = jnp.maximum(v, 0.0) if relu else v


def _tc_fin(out, b, relu):
    return pl.pallas_call(
        functools.partial(_fin_body, relu),
        grid=(GR,),
        in_specs=[_row_spec, pl.BlockSpec((1, D), lambda i: (0, 0))],
        out_specs=_row_spec,
        out_shape=jax.ShapeDtypeStruct((N, D), jnp.float32),
    )(out, b)


def kernel(x, edge_index, Ws, bs):
    src = edge_index[0]
    dst = edge_index[1]
    mask = src != dst
    pad = EPAD - E
    trash = jnp.full((pad,), TRASH, jnp.int32)
    gidx = jnp.concatenate([src, jnp.zeros((pad,), jnp.int32)]).reshape(NW, NCH, CH)
    sidx_deg = jnp.concatenate(
        [jnp.where(mask, src, TRASH), trash]).reshape(NW, NCH, CH)
    sidx = jnp.concatenate(
        [jnp.where(mask, dst, TRASH), trash]).reshape(NW, NCH, CH)
    gidx_zero = jnp.zeros((NW, NCH, CH), jnp.int32)

    ones_tab = jnp.ones((8, D), jnp.float32)
    deg_rows = _sc_prop(ones_tab, gidx_zero, sidx_deg)
    dinv = _tc_prep(deg_rows)

    h = x
    for l in range(NUM_LAYERS):
        W = Ws[l]
        out, sh = _tc_k0(h, W[0], dinv)
        z = _sc_prop(sh, gidx, sidx)
        out, tx_prev, sx_prev = _tc_step(z, h, out, W[1], dinv, 1.0, 0.0)
        tx_pp = h
        for k in range(2, K):
            z = _sc_prop(sx_prev, gidx, sidx)
            out, tx_k, sx_k = _tc_step(z, tx_pp, out, W[k], dinv, 2.0, 1.0)
            tx_pp, tx_prev, sx_prev = tx_prev, tx_k, sx_k
        h = _tc_fin(out, bs[l][None, :], relu=(l < NUM_LAYERS - 1))
    return h


# speedup vs baseline: 2.4190x; 1.0467x over previous
import functools

import jax
import jax.numpy as jnp
from jax import lax
from jax.experimental import pallas as pl
from jax.experimental.pallas import tpu as pltpu
from jax.experimental.pallas import tpu_sc as plsc

N = 10000
D = 128
E = 320000
K = 10
NUM_LAYERS = 8

NC = 2
NS = 16
NW = NC * NS
CH = 128
NCH = 80
EPT = CH * NCH
EPAD = EPT * NW
NP = 10112
RPT = NP // NS
TRASH = N

_mesh = plsc.VectorSubcoreMesh(core_axis_name="c", subcore_axis_name="s")

NBUF = 2
HALVES = 2
HNCH = NCH // HALVES


@functools.partial(
    pl.kernel,
    out_type=jax.ShapeDtypeStruct((NC, NP, D), jnp.float32),
    mesh=_mesh,
    scratch_types=[
        pltpu.VMEM((HNCH, CH), jnp.int32),
        pltpu.VMEM((HNCH, CH), jnp.int32),
        pltpu.VMEM((NBUF, CH, D), jnp.float32),
        pltpu.VMEM_SHARED((NP, D), jnp.float32),
        pltpu.SemaphoreType.DMA,
    ],
)
def _sc_prop(xs, gidx, sidx, out, gi_v, si_v, rows, acc, gsem):
    cid = lax.axis_index("c")
    sid = lax.axis_index("s")
    t = cid * NS + sid

    def _zr(i, c):
        for j in range(D // 16):
            rows[0, i, pl.ds(j * 16, 16)] = jnp.zeros((16,), jnp.float32)
        return c

    lax.fori_loop(0, CH, _zr, 0)
    for i in range(RPT // CH):
        pltpu.sync_copy(rows.at[0], acc.at[pl.ds(sid * RPT + i * CH, CH)])
    pltpu.sync_copy(rows.at[0, pl.ds(0, RPT - (RPT // CH) * CH)],
                    acc.at[pl.ds(sid * RPT + (RPT // CH) * CH,
                                 RPT - (RPT // CH) * CH)])
    plsc.subcore_barrier()

    def _gather(j, b):
        return pltpu.async_copy(xs.at[gi_v.at[j]], rows.at[b], gsem)

    def _gather_wait(j, b):
        pltpu.make_async_copy(xs.at[gi_v.at[j]], rows.at[b], gsem).wait()

    def _scatter(j, b):
        pltpu.sync_copy(rows.at[b], acc.at[si_v.at[j]], add=True)

    for h in range(HALVES):
        pltpu.sync_copy(gidx.at[t, pl.ds(h * HNCH, HNCH)], gi_v)
        pltpu.sync_copy(sidx.at[t, pl.ds(h * HNCH, HNCH)], si_v)
        _gather(0, 0)

        def _body(i, c):
            j = i * NBUF
            _gather_wait(j, 0)
            _gather(j + 1, 1)
            _scatter(j, 0)
            nj = j + 2

            @pl.when(nj < HNCH)
            def _():
                _gather(nj, 0)

            _gather_wait(j + 1, 1)
            _scatter(j + 1, 1)
            return c

        lax.fori_loop(0, HNCH // NBUF, _body, 0)

    plsc.subcore_barrier()
    pltpu.sync_copy(acc.at[pl.ds(sid * RPT, RPT)],
                    out.at[cid, pl.ds(sid * RPT, RPT)])


BR = 1000
GR = N // BR

_row_spec = pl.BlockSpec((BR, D), lambda i: (i, 0))
_w_spec = pl.BlockSpec((D, D), lambda i: (0, 0))
_z_spec = pl.BlockSpec((NC, BR, D), lambda i: (0, i, 0))


def _prep_body(z_ref, dinv_ref):
    deg = z_ref[0] + z_ref[1]
    dinv_ref[...] = jnp.where(deg > 0.0, 1.0 / jnp.sqrt(jnp.maximum(deg, 1.0)), 0.0)


def _tc_prep(z):
    return pl.pallas_call(
        _prep_body,
        grid=(GR,),
        in_specs=[_z_spec],
        out_specs=_row_spec,
        out_shape=jax.ShapeDtypeStruct((N, D), jnp.float32),
    )(z)


def _k0_body(h_ref, w_ref, dinv_ref, out_ref, sh_ref):
    h = h_ref[...]
    out_ref[...] = jnp.dot(h, w_ref[...], preferred_element_type=jnp.float32)
    sh_ref[...] = h * dinv_ref[...]


def _tc_k0(h, w0, dinv):
    return pl.pallas_call(
        _k0_body,
        grid=(GR,),
        in_specs=[_row_spec, _w_spec, _row_spec],
        out_specs=[_row_spec, _row_spec],
        out_shape=[jax.ShapeDtypeStruct((N, D), jnp.float32)] * 2,
    )(h, w0, dinv)


def _step_body(alpha, beta, z_ref, txm2_ref, out_ref, w_ref, dinv_ref,
               outn_ref, tx_ref, sx_ref):
    dinv = dinv_ref[...]
    tx = (-alpha) * dinv * (z_ref[0] + z_ref[1])
    if beta:
        tx = tx - beta * txm2_ref[...]
    outn_ref[...] = out_ref[...] + jnp.dot(tx, w_ref[...],
                                           preferred_element_type=jnp.float32)
    tx_ref[...] = tx
    sx_ref[...] = dinv * tx


def _tc_step(z, txm2, out, w, dinv, alpha, beta):
    return pl.pallas_call(
        functools.partial(_step_body, alpha, beta),
        grid=(GR,),
        in_specs=[_z_spec, _row_spec, _row_spec, _w_spec, _row_spec],
        out_specs=[_row_spec, _row_spec, _row_spec],
        out_shape=[jax.ShapeDtypeStruct((N, D), jnp.float32)] * 3,
    )(z, txm2, out, w, dinv)


def _fin_body(relu, out_ref, b_ref, h_ref):
    v = out_ref[...] + b_ref[...]
    h_ref[...] = jnp.maximum(v, 0.0) if relu else v


def _tc_fin(out, b, relu):
    return pl.pallas_call(
        functools.partial(_fin_body, relu),
        grid=(GR,),
        in_specs=[_row_spec, pl.BlockSpec((1, D), lambda i: (0, 0))],
        out_specs=_row_spec,
        out_shape=jax.ShapeDtypeStruct((N, D), jnp.float32),
    )(out, b)


def kernel(x, edge_index, Ws, bs):
    src = edge_index[0]
    dst = edge_index[1]
    mask = src != dst
    pad = EPAD - E
    trash = jnp.full((pad,), TRASH, jnp.int32)
    dst_eff = jnp.where(mask, dst, TRASH)
    perm = jnp.argsort(dst_eff)
    src_eff_deg = jnp.where(mask, src, TRASH)
    perm_deg = jnp.argsort(src_eff_deg)
    gidx = jnp.concatenate(
        [src[perm], jnp.zeros((pad,), jnp.int32)]).reshape(NW, NCH, CH)
    sidx = jnp.concatenate([dst_eff[perm], trash]).reshape(NW, NCH, CH)
    sidx_deg = jnp.concatenate(
        [src_eff_deg[perm_deg], trash]).reshape(NW, NCH, CH)
    gidx_zero = jnp.zeros((NW, NCH, CH), jnp.int32)

    ones_tab = jnp.ones((8, D), jnp.float32)
    deg_rows = _sc_prop(ones_tab, gidx_zero, sidx_deg)
    dinv = _tc_prep(deg_rows)

    h = x
    for l in range(NUM_LAYERS):
        W = Ws[l]
        out, sh = _tc_k0(h, W[0], dinv)
        z = _sc_prop(sh, gidx, sidx)
        out, tx_prev, sx_prev = _tc_step(z, h, out, W[1], dinv, 1.0, 0.0)
        tx_pp = h
        for k in range(2, K):
            z = _sc_prop(sx_prev, gidx, sidx)
            out, tx_k, sx_k = _tc_step(z, tx_pp, out, W[k], dinv, 2.0, 1.0)
            tx_pp, tx_prev, sx_prev = tx_prev, tx_k, sx_k
        h = _tc_fin(out, bs[l][None, :], relu=(l < NUM_LAYERS - 1))
    return h


# speedup vs baseline: 3.2273x; 1.3342x over previous
import functools

import jax
import jax.numpy as jnp
from jax import lax
from jax.experimental import pallas as pl
from jax.experimental.pallas import tpu as pltpu
from jax.experimental.pallas import tpu_sc as plsc

N = 10000
D = 128
E = 320000
K = 10
NUM_LAYERS = 8

NC = 2
NS = 16
NW = NC * NS
CH = 128
NCH = 80
EPT = CH * NCH
EPAD = EPT * NW
NP = 10112
RPT = NP // NS
TRASH = N

_mesh = plsc.VectorSubcoreMesh(core_axis_name="c", subcore_axis_name="s")

NBUF = 2
HALVES = 2
HNCH = NCH // HALVES


@functools.partial(
    pl.kernel,
    out_type=jax.ShapeDtypeStruct((NC, NP, D), jnp.float32),
    mesh=_mesh,
    scratch_types=[
        pltpu.VMEM((HNCH, CH), jnp.int32),
        pltpu.VMEM((HNCH, CH), jnp.int32),
        pltpu.VMEM((NBUF, CH, D), jnp.float32),
        pltpu.VMEM_SHARED((NP, D), jnp.float32),
        pltpu.SemaphoreType.DMA,
        pltpu.SemaphoreType.DMA,
    ],
)
def _sc_prop(xs, gidx, sidx, out, gi_v, si_v, rows, acc, gsem, ssem):
    cid = lax.axis_index("c")
    sid = lax.axis_index("s")
    t = cid * NS + sid

    def _zr(i, c):
        for j in range(D // 16):
            rows[0, i, pl.ds(j * 16, 16)] = jnp.zeros((16,), jnp.float32)
        return c

    lax.fori_loop(0, CH, _zr, 0)
    for i in range(RPT // CH):
        pltpu.sync_copy(rows.at[0], acc.at[pl.ds(sid * RPT + i * CH, CH)])
    pltpu.sync_copy(rows.at[0, pl.ds(0, RPT - (RPT // CH) * CH)],
                    acc.at[pl.ds(sid * RPT + (RPT // CH) * CH,
                                 RPT - (RPT // CH) * CH)])
    plsc.subcore_barrier()

    def _gather(j, b):
        return pltpu.async_copy(xs.at[gi_v.at[j]], rows.at[b], gsem)

    def _gather_wait(j, b):
        pltpu.make_async_copy(xs.at[gi_v.at[j]], rows.at[b], gsem).wait()

    def _scatter(j, b):
        return pltpu.async_copy(rows.at[b], acc.at[si_v.at[j]], ssem, add=True)

    for h in range(HALVES):
        pltpu.sync_copy(gidx.at[t, pl.ds(h * HNCH, HNCH)], gi_v)
        pltpu.sync_copy(sidx.at[t, pl.ds(h * HNCH, HNCH)], si_v)
        _gather(0, 0)
        _gather(1, 1)

        def _body(i, c):
            j = i * NBUF
            _gather_wait(j, 0)
            sa = _scatter(j, 0)
            _gather_wait(j + 1, 1)
            sb = _scatter(j + 1, 1)
            sa.wait()

            @pl.when(j + 2 < HNCH)
            def _():
                _gather(j + 2, 0)

            sb.wait()

            @pl.when(j + 3 < HNCH)
            def _():
                _gather(j + 3, 1)

            return c

        lax.fori_loop(0, HNCH // NBUF, _body, 0)

    plsc.subcore_barrier()
    pltpu.sync_copy(acc.at[pl.ds(sid * RPT, RPT)],
                    out.at[cid, pl.ds(sid * RPT, RPT)])


BR = 1000
GR = N // BR

_row_spec = pl.BlockSpec((BR, D), lambda i: (i, 0))
_w_spec = pl.BlockSpec((D, D), lambda i: (0, 0))
_z_spec = pl.BlockSpec((NC, BR, D), lambda i: (0, i, 0))


def _prep_body(z_ref, dinv_ref):
    deg = z_ref[0] + z_ref[1]
    dinv_ref[...] = jnp.where(deg > 0.0, 1.0 / jnp.sqrt(jnp.maximum(deg, 1.0)), 0.0)


def _tc_prep(z):
    return pl.pallas_call(
        _prep_body,
        grid=(GR,),
        in_specs=[_z_spec],
        out_specs=_row_spec,
        out_shape=jax.ShapeDtypeStruct((N, D), jnp.float32),
    )(z)


def _k0_body(h_ref, w_ref, dinv_ref, out_ref, sh_ref):
    h = h_ref[...]
    out_ref[...] = jnp.dot(h, w_ref[...], preferred_element_type=jnp.float32)
    sh_ref[...] = h * dinv_ref[...]


def _tc_k0(h, w0, dinv):
    return pl.pallas_call(
        _k0_body,
        grid=(GR,),
        in_specs=[_row_spec, _w_spec, _row_spec],
        out_specs=[_row_spec, _row_spec],
        out_shape=[jax.ShapeDtypeStruct((N, D), jnp.float32)] * 2,
    )(h, w0, dinv)


def _step_body(alpha, beta, z_ref, txm2_ref, out_ref, w_ref, dinv_ref,
               outn_ref, tx_ref, sx_ref):
    dinv = dinv_ref[...]
    tx = (-alpha) * dinv * (z_ref[0] + z_ref[1])
    if beta:
        tx = tx - beta * txm2_ref[...]
    outn_ref[...] = out_ref[...] + jnp.dot(tx, w_ref[...],
                                           preferred_element_type=jnp.float32)
    tx_ref[...] = tx
    sx_ref[...] = dinv * tx


def _tc_step(z, txm2, out, w, dinv, alpha, beta):
    return pl.pallas_call(
        functools.partial(_step_body, alpha, beta),
        grid=(GR,),
        in_specs=[_z_spec, _row_spec, _row_spec, _w_spec, _row_spec],
        out_specs=[_row_spec, _row_spec, _row_spec],
        out_shape=[jax.ShapeDtypeStruct((N, D), jnp.float32)] * 3,
    )(z, txm2, out, w, dinv)


def _fin_body(relu, out_ref, b_ref, h_ref):
    v = out_ref[...] + b_ref[...]
    h_ref[...] = jnp.maximum(v, 0.0) if relu else v


def _tc_fin(out, b, relu):
    return pl.pallas_call(
        functools.partial(_fin_body, relu),
        grid=(GR,),
        in_specs=[_row_spec, pl.BlockSpec((1, D), lambda i: (0, 0))],
        out_specs=_row_spec,
        out_shape=jax.ShapeDtypeStruct((N, D), jnp.float32),
    )(out, b)


def kernel(x, edge_index, Ws, bs):
    src = edge_index[0]
    dst = edge_index[1]
    mask = src != dst
    pad = EPAD - E
    trash = jnp.full((pad,), TRASH, jnp.int32)
    dst_eff = jnp.where(mask, dst, TRASH)
    perm = jnp.argsort(dst_eff)
    src_eff_deg = jnp.where(mask, src, TRASH)
    perm_deg = jnp.argsort(src_eff_deg)
    gidx = jnp.concatenate(
        [src[perm], jnp.zeros((pad,), jnp.int32)]).reshape(NW, NCH, CH)
    sidx = jnp.concatenate([dst_eff[perm], trash]).reshape(NW, NCH, CH)
    sidx_deg = jnp.concatenate(
        [src_eff_deg[perm_deg], trash]).reshape(NW, NCH, CH)
    gidx_ones = (jnp.arange(EPAD, dtype=jnp.int32) % 1024).reshape(NW, NCH, CH)

    ones_tab = jnp.ones((1024, D), jnp.float32)
    deg_rows = _sc_prop(ones_tab, gidx_ones, sidx_deg)
    dinv = _tc_prep(deg_rows)

    h = x
    for l in range(NUM_LAYERS):
        W = Ws[l]
        out, sh = _tc_k0(h, W[0], dinv)
        z = _sc_prop(sh, gidx, sidx)
        out, tx_prev, sx_prev = _tc_step(z, h, out, W[1], dinv, 1.0, 0.0)
        tx_pp = h
        for k in range(2, K):
            z = _sc_prop(sx_prev, gidx, sidx)
            out, tx_k, sx_k = _tc_step(z, tx_pp, out, W[k], dinv, 2.0, 1.0)
            tx_pp, tx_prev, sx_prev = tx_prev, tx_k, sx_k
        h = _tc_fin(out, bs[l][None, :], relu=(l < NUM_LAYERS - 1))
    return h
